# baseline (speedup 1.0000x reference)
import functools

import jax
import jax.numpy as jnp
from jax import lax
from jax.experimental import pallas as pl
from jax.experimental.pallas import tpu as pltpu
from jax.experimental.pallas import tpu_sc as plsc

N = 10000
D = 128
NP = 10240
NC = 2
NS = 16
RPT = NP // NS
CH = 128
NB = 2
ZR = 64
DEGW = 16
DH = D // NC


def _make_sc_agg(chunks: int):
    assert chunks % NB == 0
    mesh = plsc.VectorSubcoreMesh(core_axis_name="c", subcore_axis_name="s")

    @functools.partial(
        pl.kernel,
        mesh=mesh,
        compiler_params=pltpu.CompilerParams(use_tc_tiling_on_sc=False),
        out_type=[
            jax.ShapeDtypeStruct((NP, D), jnp.float32),
            jax.ShapeDtypeStruct((NC, NP, DEGW), jnp.float32),
        ],
        scratch_types=[
            pltpu.VMEM((chunks // 2, 2 * CH), jnp.int32),
            pltpu.VMEM((chunks, CH), jnp.int32),
            [pltpu.VMEM((2 * CH, DH), jnp.float32)] * NB,
            pltpu.VMEM((CH, DEGW), jnp.float32),
            pltpu.VMEM((ZR, DEGW), jnp.float32),
            pltpu.VMEM_SHARED((NP, DH), jnp.float32),
            pltpu.VMEM_SHARED((NP, DEGW), jnp.float32),
            [pltpu.SemaphoreType.DMA] * NB,
        ],
    )
    def sc_agg(xh_hbm, src_hbm, dst_hbm, agg_out, deg_out,
               src_v, dst_v, rows, ones_v, zdeg_v,
               agg_s, deg_s, sem_g):
        cid = lax.axis_index("c")
        sid = lax.axis_index("s")
        base = sid * RPT

        zero16 = jnp.zeros((16,), jnp.float32)
        one16 = jnp.ones((16,), jnp.float32)

        def fill_zrow(i, _):
            for g in range(DH // 16):
                rows[0][i, pl.ds(g * 16, 16)] = zero16
            return 0
        lax.fori_loop(0, 2 * CH, fill_zrow, 0)

        def fill_zdeg(i, _):
            zdeg_v[i, :] = zero16
            return 0
        lax.fori_loop(0, ZR, fill_zdeg, 0)

        def fill_ones(i, _):
            ones_v[i, :] = one16
            return 0
        lax.fori_loop(0, CH, fill_ones, 0)

        for j in range(RPT // (2 * CH)):
            pltpu.sync_copy(rows[0],
                            agg_s.at[pl.ds(base + j * 2 * CH, 2 * CH)])
        if RPT % (2 * CH):
            pltpu.sync_copy(
                rows[0].at[pl.ds(0, RPT % (2 * CH))],
                agg_s.at[pl.ds(base + (RPT // (2 * CH)) * 2 * CH,
                               RPT % (2 * CH))])
        for j in range(RPT // ZR):
            pltpu.sync_copy(zdeg_v, deg_s.at[pl.ds(base + j * ZR, ZR)])

        pltpu.sync_copy(src_hbm.at[sid], src_v)
        pltpu.sync_copy(dst_hbm.at[sid], dst_v)
        plsc.subcore_barrier()

        npairs = chunks // 2

        def gather(p, b):
            pltpu.async_copy(xh_hbm.at[cid].at[src_v.at[p]], rows[b],
                             sem_g[b])

        def wait_gather(p, b):
            pltpu.make_async_copy(xh_hbm.at[cid].at[src_v.at[p]], rows[b],
                                  sem_g[b]).wait()

        def scatter_pair(p, b):
            pltpu.sync_copy(rows[b].at[pl.ds(0, CH)],
                            agg_s.at[dst_v.at[2 * p]], add=True)

            @pl.when(cid == 0)
            def _():
                pltpu.sync_copy(ones_v, deg_s.at[dst_v.at[2 * p]], add=True)
            pltpu.sync_copy(rows[b].at[pl.ds(CH, CH)],
                            agg_s.at[dst_v.at[2 * p + 1]], add=True)

            @pl.when(cid == 1)
            def _():
                pltpu.sync_copy(ones_v, deg_s.at[dst_v.at[2 * p + 1]],
                                add=True)

        gather(0, 0)

        def pair_body(g, _):
            p0 = NB * g
            for b in range(NB):
                p = p0 + b

                @pl.when(p + 1 < npairs)
                def _():
                    gather(p + 1, (b + 1) % NB)
                wait_gather(p, b)
                scatter_pair(p, b)
            return 0
        lax.fori_loop(0, npairs // NB, pair_body, 0)
        plsc.subcore_barrier()

        pltpu.sync_copy(agg_s.at[pl.ds(base, RPT)],
                        agg_out.at[pl.ds(base, RPT), pl.ds(cid * DH, DH)])
        pltpu.sync_copy(deg_s.at[pl.ds(base, RPT)],
                        deg_out.at[cid, pl.ds(base, RPT)])

    return sc_agg


def _tc_loop_msg(x_ref, w_ref, c_ref, o_ref):
    c = c_ref[0, 0]
    sc = jnp.sqrt(c)
    xb = x_ref[...]
    x_norm = jnp.maximum(
        jnp.sqrt(jnp.sum(xb * xb, axis=1, keepdims=True)), 1e-5)
    mx = jnp.dot(xb, w_ref[...], preferred_element_type=jnp.float32)
    mx_norm = jnp.maximum(
        jnp.sqrt(jnp.sum(mx * mx, axis=1, keepdims=True)), 1e-5)
    a = jnp.clip(sc * x_norm, -1.0 + 1e-7, 1.0 - 1e-7)
    artanh = 0.5 * jnp.log((1.0 + a) / (1.0 - a))
    o_ref[...] = jnp.tanh(mx_norm / x_norm * artanh) * mx / (mx_norm * sc)


def _tc_combine(agg_ref, deg_ref, lm_ref, b_ref, c_ref, o_ref):
    c = c_ref[0, 0]
    deg = (deg_ref[0] + deg_ref[1])[:, 0:1]
    h = agg_ref[...] / jnp.maximum(deg, 1.0)

    def mobius_add(u, v):
        u2 = jnp.sum(u * u, axis=-1, keepdims=True)
        v2 = jnp.sum(v * v, axis=-1, keepdims=True)
        uv = jnp.sum(u * v, axis=-1, keepdims=True)
        num = (1.0 + 2.0 * c * uv + c * v2) * u + (1.0 - c * u2) * v
        den = 1.0 + 2.0 * c * uv + c * c * u2 * v2
        return num / (den + 1e-15)

    h = mobius_add(h, b_ref[...])
    h = mobius_add(h, lm_ref[...])
    o_ref[...] = jnp.maximum(h, 0.0)


def kernel(x, edge_index, loop_weight, bias, k, reverse):
    E = edge_index.shape[1]
    epe = NS * CH * 2 * NB
    e_pad = ((E + epe - 1) // epe) * epe
    chunks = e_pad // (NS * CH)

    src = jnp.where(reverse, edge_index[1], edge_index[0]).astype(jnp.int32)
    dst = jnp.where(reverse, edge_index[0], edge_index[1]).astype(jnp.int32)
    pad = e_pad - E
    src = jnp.concatenate([src, jnp.zeros((pad,), jnp.int32)])
    dst = jnp.concatenate([dst, jnp.full((pad,), NP - 1, jnp.int32)])
    src3 = src.reshape(NS, chunks // 2, 2 * CH)
    dst3 = dst.reshape(NS, chunks, CH)

    xh = jnp.stack([x[:, :DH], x[:, DH:]], axis=0)

    agg, deg = _make_sc_agg(chunks)(xh, src3, dst3)

    blk = 2000
    c11 = k.reshape(1, 1).astype(jnp.float32)
    loop_msg = pl.pallas_call(
        _tc_loop_msg,
        grid=(N // blk,),
        in_specs=[
            pl.BlockSpec((blk, D), lambda i: (i, 0)),
            pl.BlockSpec((D, D), lambda i: (0, 0)),
            pl.BlockSpec(memory_space=pltpu.SMEM),
        ],
        out_specs=pl.BlockSpec((blk, D), lambda i: (i, 0)),
        out_shape=jax.ShapeDtypeStruct((N, D), jnp.float32),
    )(x, loop_weight, c11)

    out = pl.pallas_call(
        _tc_combine,
        grid=(N // blk,),
        in_specs=[
            pl.BlockSpec((blk, D), lambda i: (i, 0)),
            pl.BlockSpec((NC, blk, DEGW), lambda i: (0, i, 0)),
            pl.BlockSpec((blk, D), lambda i: (i, 0)),
            pl.BlockSpec((1, D), lambda i: (0, 0)),
            pl.BlockSpec(memory_space=pltpu.SMEM),
        ],
        out_specs=pl.BlockSpec((blk, D), lambda i: (i, 0)),
        out_shape=jax.ShapeDtypeStruct((N, D), jnp.float32),
    )(agg, deg, loop_msg, bias.reshape(1, D), c11)
    return out


# speedup vs baseline: 1.3084x; 1.3084x over previous
import functools

import jax
import jax.numpy as jnp
from jax import lax
from jax.experimental import pallas as pl
from jax.experimental.pallas import tpu as pltpu
from jax.experimental.pallas import tpu_sc as plsc

N = 10000
D = 128
NP = 10240
NC = 2
NS = 16
RPT = NP // NS
CH = 128
NB = 2
ZR = 64
DEGW = 16
DH = D // NC


def _make_sc_agg(chunks: int):
    assert chunks % NB == 0
    mesh = plsc.VectorSubcoreMesh(core_axis_name="c", subcore_axis_name="s")

    @functools.partial(
        pl.kernel,
        mesh=mesh,
        compiler_params=pltpu.CompilerParams(use_tc_tiling_on_sc=False),
        out_type=[
            jax.ShapeDtypeStruct((NP, D), jnp.float32),
            jax.ShapeDtypeStruct((NC, NP, DEGW), jnp.float32),
        ],
        scratch_types=[
            pltpu.VMEM((chunks, CH), jnp.int32),
            pltpu.VMEM((chunks, CH), jnp.int32),
            [pltpu.VMEM((CH, DH), jnp.float32)] * NB,
            pltpu.VMEM((CH, DEGW), jnp.float32),
            pltpu.VMEM((ZR, DEGW), jnp.float32),
            pltpu.VMEM_SHARED((NP, DH), jnp.float32),
            pltpu.VMEM_SHARED((NP, DEGW), jnp.float32),
            [pltpu.SemaphoreType.DMA] * NB,
        ],
    )
    def sc_agg(xh_hbm, src_hbm, dst_hbm, agg_out, deg_out,
               src_v, dst_v, rows, ones_v, zdeg_v,
               agg_s, deg_s, sem_g):
        cid = lax.axis_index("c")
        sid = lax.axis_index("s")
        base = sid * RPT

        zero16 = jnp.zeros((16,), jnp.float32)
        one16 = jnp.ones((16,), jnp.float32)

        def fill_zrow(i, _):
            for g in range(DH // 16):
                rows[0][i, pl.ds(g * 16, 16)] = zero16
            return 0
        lax.fori_loop(0, CH, fill_zrow, 0)

        def fill_zdeg(i, _):
            zdeg_v[i, :] = zero16
            return 0
        lax.fori_loop(0, ZR, fill_zdeg, 0)

        def fill_ones(i, _):
            ones_v[i, :] = one16
            return 0
        lax.fori_loop(0, CH, fill_ones, 0)

        for j in range(RPT // CH):
            pltpu.sync_copy(rows[0], agg_s.at[pl.ds(base + j * CH, CH)])
        for j in range(RPT // ZR):
            pltpu.sync_copy(zdeg_v, deg_s.at[pl.ds(base + j * ZR, ZR)])

        pltpu.sync_copy(src_hbm.at[sid], src_v)
        pltpu.sync_copy(dst_hbm.at[sid], dst_v)
        plsc.subcore_barrier()

        def gather(j, b):
            pltpu.async_copy(xh_hbm.at[cid].at[src_v.at[j]], rows[b],
                             sem_g[b])

        def wait_gather(j, b):
            pltpu.make_async_copy(xh_hbm.at[cid].at[src_v.at[j]], rows[b],
                                  sem_g[b]).wait()

        gather(0, 0)

        def pair_body(g, _):
            j0 = 2 * g
            gather(j0 + 1, 1)
            wait_gather(j0, 0)
            pltpu.sync_copy(rows[0], agg_s.at[dst_v.at[j0]], add=True)

            @pl.when(cid == 0)
            def _():
                pltpu.sync_copy(ones_v, deg_s.at[dst_v.at[j0]], add=True)

            @pl.when(j0 + 2 < chunks)
            def _():
                gather(j0 + 2, 0)
            wait_gather(j0 + 1, 1)
            pltpu.sync_copy(rows[1], agg_s.at[dst_v.at[j0 + 1]], add=True)

            @pl.when(cid == 1)
            def _():
                pltpu.sync_copy(ones_v, deg_s.at[dst_v.at[j0 + 1]],
                                add=True)
            return 0
        lax.fori_loop(0, chunks // 2, pair_body, 0)
        plsc.subcore_barrier()

        pltpu.sync_copy(agg_s.at[pl.ds(base, RPT)],
                        agg_out.at[pl.ds(base, RPT), pl.ds(cid * DH, DH)])
        pltpu.sync_copy(deg_s.at[pl.ds(base, RPT)],
                        deg_out.at[cid, pl.ds(base, RPT)])

    return sc_agg


def _tc_loop_msg(x_ref, w_ref, c_ref, o_ref):
    c = c_ref[0, 0]
    sc = jnp.sqrt(c)
    xb = x_ref[...]
    x_norm = jnp.maximum(
        jnp.sqrt(jnp.sum(xb * xb, axis=1, keepdims=True)), 1e-5)
    mx = jnp.dot(xb, w_ref[...], preferred_element_type=jnp.float32)
    mx_norm = jnp.maximum(
        jnp.sqrt(jnp.sum(mx * mx, axis=1, keepdims=True)), 1e-5)
    a = jnp.clip(sc * x_norm, -1.0 + 1e-7, 1.0 - 1e-7)
    artanh = 0.5 * jnp.log((1.0 + a) / (1.0 - a))
    o_ref[...] = jnp.tanh(mx_norm / x_norm * artanh) * mx / (mx_norm * sc)


def _tc_combine(agg_ref, deg_ref, lm_ref, b_ref, c_ref, o_ref):
    c = c_ref[0, 0]
    deg = (deg_ref[0] + deg_ref[1])[:, 0:1]
    h = agg_ref[...] / jnp.maximum(deg, 1.0)

    def mobius_add(u, v):
        u2 = jnp.sum(u * u, axis=-1, keepdims=True)
        v2 = jnp.sum(v * v, axis=-1, keepdims=True)
        uv = jnp.sum(u * v, axis=-1, keepdims=True)
        num = (1.0 + 2.0 * c * uv + c * v2) * u + (1.0 - c * u2) * v
        den = 1.0 + 2.0 * c * uv + c * c * u2 * v2
        return num / (den + 1e-15)

    h = mobius_add(h, b_ref[...])
    h = mobius_add(h, lm_ref[...])
    o_ref[...] = jnp.maximum(h, 0.0)


def kernel(x, edge_index, loop_weight, bias, k, reverse):
    E = edge_index.shape[1]
    epe = NS * CH * 2
    e_pad = ((E + epe - 1) // epe) * epe
    chunks = e_pad // (NS * CH)

    src = jnp.where(reverse, edge_index[1], edge_index[0]).astype(jnp.int32)
    dst = jnp.where(reverse, edge_index[0], edge_index[1]).astype(jnp.int32)
    pad = e_pad - E
    src = jnp.concatenate([src, jnp.zeros((pad,), jnp.int32)])
    dst = jnp.concatenate([dst, jnp.full((pad,), NP - 1, jnp.int32)])
    src3 = src.reshape(NS, chunks, CH)
    dst3 = dst.reshape(NS, chunks, CH)

    xh = jnp.stack([x[:, :DH], x[:, DH:]], axis=0)

    agg, deg = _make_sc_agg(chunks)(xh, src3, dst3)

    blk = 2000
    c11 = k.reshape(1, 1).astype(jnp.float32)
    loop_msg = pl.pallas_call(
        _tc_loop_msg,
        grid=(N // blk,),
        in_specs=[
            pl.BlockSpec((blk, D), lambda i: (i, 0)),
            pl.BlockSpec((D, D), lambda i: (0, 0)),
            pl.BlockSpec(memory_space=pltpu.SMEM),
        ],
        out_specs=pl.BlockSpec((blk, D), lambda i: (i, 0)),
        out_shape=jax.ShapeDtypeStruct((N, D), jnp.float32),
    )(x, loop_weight, c11)

    out = pl.pallas_call(
        _tc_combine,
        grid=(N // blk,),
        in_specs=[
            pl.BlockSpec((blk, D), lambda i: (i, 0)),
            pl.BlockSpec((NC, blk, DEGW), lambda i: (0, i, 0)),
            pl.BlockSpec((blk, D), lambda i: (i, 0)),
            pl.BlockSpec((1, D), lambda i: (0, 0)),
            pl.BlockSpec(memory_space=pltpu.SMEM),
        ],
        out_specs=pl.BlockSpec((blk, D), lambda i: (i, 0)),
        out_shape=jax.ShapeDtypeStruct((N, D), jnp.float32),
    )(agg, deg, loop_msg, bias.reshape(1, D), c11)
    return out
